# baseline (device time: 18284 ns/iter reference)
import jax
import jax.numpy as jnp
from jax import lax
from jax.experimental import pallas as pl
from jax.experimental.pallas import tpu as pltpu

N_DEV = 8
BLK = 256

_ORDER = [0, 4, 1, 7, 3, 5, 2, 6]
_CHUNKS = [(0, 1), (2, 3), (4, 5), (6, 7)]


def kernel(x, w_mat):
    k, m_per = x.shape
    kw, n = w_mat.shape
    assert m_per == BLK and k == N_DEV * BLK

    def body(x_hbm, w_hbm, out_hbm, xv_ref, xb_ref, gx_ref, wbuf_ref, acc_ref,
             send_sems, recv_sems, w_sems, x_sem, out_sems, credit_sem):
        my = lax.axis_index("i")

        pltpu.make_async_copy(x_hbm, xv_ref, x_sem).start()

        def start_w_chunk(c, slot):
            for h, t in enumerate(_CHUNKS[c]):
                j = (my - _ORDER[t]) % N_DEV
                pltpu.make_async_copy(
                    w_hbm.at[pl.ds(j * BLK, BLK), :],
                    wbuf_ref.at[slot, pl.ds(h * BLK, BLK), :],
                    w_sems.at[slot, h],
                ).start()

        def wait_w_chunk(c, slot):
            for h, t in enumerate(_CHUNKS[c]):
                j = (my - _ORDER[t]) % N_DEV
                pltpu.make_async_copy(
                    w_hbm.at[pl.ds(j * BLK, BLK), :],
                    wbuf_ref.at[slot, pl.ds(h * BLK, BLK), :],
                    w_sems.at[slot, h],
                ).wait()

        start_w_chunk(0, 0)

        barrier = pltpu.get_barrier_semaphore()
        for d in range(1, N_DEV):
            peer = (my + d) % N_DEV
            pl.semaphore_signal(
                barrier, inc=1,
                device_id=(peer,), device_id_type=pl.DeviceIdType.MESH,
            )

        pltpu.make_async_copy(x_hbm, xv_ref, x_sem).wait()
        xb_ref[:, :] = xv_ref[:, :].astype(jnp.bfloat16)

        pl.semaphore_wait(barrier, N_DEV - 1)

        sends = []
        for t in range(1, N_DEV):
            d = _ORDER[t]
            peer = (my + d) % N_DEV
            rdma = pltpu.make_async_remote_copy(
                src_ref=xb_ref.at[pl.ds(peer * BLK, BLK), :],
                dst_ref=gx_ref.at[:, pl.ds(t * BLK, BLK)],
                send_sem=send_sems.at[d - 1],
                recv_sem=recv_sems.at[d - 1],
                device_id=(peer,),
                device_id_type=pl.DeviceIdType.MESH,
            )
            rdma.start()
            sends.append(rdma)

        gx_ref[:, pl.ds(0, BLK)] = xb_ref[pl.ds(my * BLK, BLK), :]

        n_chunks = len(_CHUNKS)
        for c, chunk in enumerate(_CHUNKS):
            slot = c % 2
            if c + 1 < n_chunks:
                start_w_chunk(c + 1, 1 - slot)
            wait_w_chunk(c, slot)
            for t in chunk:
                d = _ORDER[t]
                if d == 0:
                    continue
                recv = pltpu.make_async_remote_copy(
                    src_ref=xb_ref.at[pl.ds(0, BLK), :],
                    dst_ref=gx_ref.at[:, pl.ds(t * BLK, BLK)],
                    send_sem=send_sems.at[d - 1],
                    recv_sem=recv_sems.at[d - 1],
                    device_id=(my,),
                    device_id_type=pl.DeviceIdType.MESH,
                )
                recv.wait_recv()
            kblk = len(chunk) * BLK
            if c + 1 < n_chunks:
                part = jnp.dot(
                    gx_ref[:, pl.ds(chunk[0] * BLK, kblk)],
                    wbuf_ref[slot, pl.ds(0, kblk), :],
                    preferred_element_type=jnp.float32,
                )
                if c == 0:
                    acc_ref[:, :] = part
                else:
                    acc_ref[:, :] = acc_ref[:, :] + part
            else:
                half = (N_DEV * BLK) // 2
                for g in range(2):
                    part = jnp.dot(
                        gx_ref[:, pl.ds(chunk[0] * BLK, kblk)],
                        wbuf_ref[slot, pl.ds(0, kblk), pl.ds(g * half, half)],
                        preferred_element_type=jnp.float32,
                    )
                    y = acc_ref[:, pl.ds(g * half, half)] + part
                    acc_ref[:, pl.ds(g * half, half)] = y * jax.nn.sigmoid(y)
                    pltpu.make_async_copy(
                        acc_ref.at[:, pl.ds(g * half, half)],
                        out_hbm.at[:, pl.ds(g * half, half)],
                        out_sems.at[g],
                    ).start()
            for t in chunk:
                d = _ORDER[t]
                if d == 0:
                    continue
                src = (my - d) % N_DEV
                pl.semaphore_signal(
                    credit_sem, inc=1,
                    device_id=(src,), device_id_type=pl.DeviceIdType.MESH,
                )

        for rdma in sends:
            rdma.wait_send()
        half = (N_DEV * BLK) // 2
        for g in range(2):
            pltpu.make_async_copy(
                acc_ref.at[:, pl.ds(g * half, half)],
                out_hbm.at[:, pl.ds(g * half, half)],
                out_sems.at[g],
            ).wait()

        pl.semaphore_wait(credit_sem, N_DEV - 1)

    x = pltpu.with_memory_space_constraint(x, pltpu.MemorySpace.HBM)
    w_mat = pltpu.with_memory_space_constraint(w_mat, pltpu.MemorySpace.HBM)
    return pl.pallas_call(
        body,
        out_shape=jax.ShapeDtypeStruct((BLK, n), jnp.float32),
        in_specs=[
            pl.BlockSpec(memory_space=pltpu.MemorySpace.HBM),
            pl.BlockSpec(memory_space=pltpu.MemorySpace.HBM),
        ],
        out_specs=pl.BlockSpec(memory_space=pltpu.MemorySpace.HBM),
        scratch_shapes=[
            pltpu.VMEM((N_DEV * BLK, BLK), jnp.float32),
            pltpu.VMEM((N_DEV * BLK, BLK), jnp.bfloat16),
            pltpu.VMEM((BLK, N_DEV * BLK), jnp.bfloat16),
            pltpu.VMEM((2, 2 * BLK, N_DEV * BLK), jnp.float32),
            pltpu.VMEM((BLK, N_DEV * BLK), jnp.float32),
            pltpu.SemaphoreType.DMA((N_DEV - 1,)),
            pltpu.SemaphoreType.DMA((N_DEV - 1,)),
            pltpu.SemaphoreType.DMA((2, 2)),
            pltpu.SemaphoreType.DMA,
            pltpu.SemaphoreType.DMA((2,)),
            pltpu.SemaphoreType.REGULAR,
        ],
        compiler_params=pltpu.CompilerParams(collective_id=0),
    )(x, w_mat)


# device time: 17960 ns/iter; 1.0180x vs baseline; 1.0180x over previous
import jax
import jax.numpy as jnp
from jax import lax
from jax.experimental import pallas as pl
from jax.experimental.pallas import tpu as pltpu

N_DEV = 8
BLK = 256

_ORDER = [0, 4, 1, 7, 3, 5, 2, 6]
_CHUNKS = [(0, 1), (2, 3), (4, 5), (6, 7)]


def kernel(x, w_mat):
    k, m_per = x.shape
    kw, n = w_mat.shape
    assert m_per == BLK and k == N_DEV * BLK

    def body(x_hbm, w_hbm, out_hbm, xv_ref, xb_ref, gx_ref, wbuf_ref, acc_ref,
             send_sems, recv_sems, w_sems, x_sem, out_sems, credit_sem):
        my = lax.axis_index("i")

        pltpu.make_async_copy(x_hbm, xv_ref, x_sem).start()

        def start_w_chunk(c, slot):
            for h, t in enumerate(_CHUNKS[c]):
                j = (my - _ORDER[t]) % N_DEV
                pltpu.make_async_copy(
                    w_hbm.at[pl.ds(j * BLK, BLK), :],
                    wbuf_ref.at[slot, pl.ds(h * BLK, BLK), :],
                    w_sems.at[slot, h],
                ).start()

        def wait_w_chunk(c, slot):
            for h, t in enumerate(_CHUNKS[c]):
                j = (my - _ORDER[t]) % N_DEV
                pltpu.make_async_copy(
                    w_hbm.at[pl.ds(j * BLK, BLK), :],
                    wbuf_ref.at[slot, pl.ds(h * BLK, BLK), :],
                    w_sems.at[slot, h],
                ).wait()

        start_w_chunk(0, 0)

        barrier = pltpu.get_barrier_semaphore()
        for d in range(1, N_DEV):
            peer = (my + d) % N_DEV
            pl.semaphore_signal(
                barrier, inc=1,
                device_id=(peer,), device_id_type=pl.DeviceIdType.MESH,
            )

        pltpu.make_async_copy(x_hbm, xv_ref, x_sem).wait()
        xb_ref[:, :] = xv_ref[:, :].astype(jnp.bfloat16)

        pl.semaphore_wait(barrier, N_DEV - 1)

        sends = []
        for t in range(1, N_DEV):
            d = _ORDER[t]
            peer = (my + d) % N_DEV
            rdma = pltpu.make_async_remote_copy(
                src_ref=xb_ref.at[pl.ds(peer * BLK, BLK), :],
                dst_ref=gx_ref.at[:, pl.ds(t * BLK, BLK)],
                send_sem=send_sems.at[d - 1],
                recv_sem=recv_sems.at[d - 1],
                device_id=(peer,),
                device_id_type=pl.DeviceIdType.MESH,
            )
            rdma.start()
            sends.append(rdma)

        gx_ref[:, pl.ds(0, BLK)] = xb_ref[pl.ds(my * BLK, BLK), :]

        n_chunks = len(_CHUNKS)
        for c, chunk in enumerate(_CHUNKS):
            slot = c % 2
            if c + 1 < n_chunks:
                start_w_chunk(c + 1, 1 - slot)
            wait_w_chunk(c, slot)
            for t in chunk:
                d = _ORDER[t]
                if d == 0:
                    continue
                recv = pltpu.make_async_remote_copy(
                    src_ref=xb_ref.at[pl.ds(0, BLK), :],
                    dst_ref=gx_ref.at[:, pl.ds(t * BLK, BLK)],
                    send_sem=send_sems.at[d - 1],
                    recv_sem=recv_sems.at[d - 1],
                    device_id=(my,),
                    device_id_type=pl.DeviceIdType.MESH,
                )
                recv.wait_recv()
            kblk = len(chunk) * BLK
            if c + 1 < n_chunks:
                part = jnp.dot(
                    gx_ref[:, pl.ds(chunk[0] * BLK, kblk)],
                    wbuf_ref[slot, pl.ds(0, kblk), :],
                    preferred_element_type=jnp.float32,
                )
                if c == 0:
                    acc_ref[:, :] = part
                else:
                    acc_ref[:, :] = acc_ref[:, :] + part
            else:
                part = jnp.dot(
                    gx_ref[:, pl.ds(chunk[0] * BLK, kblk)],
                    wbuf_ref[slot, pl.ds(0, kblk), :],
                    preferred_element_type=jnp.float32,
                )
                y = acc_ref[:, :] + part
                acc_ref[:, :] = y * jax.nn.sigmoid(y)
                pltpu.make_async_copy(acc_ref, out_hbm, out_sems.at[0]).start()
            for t in chunk:
                d = _ORDER[t]
                if d == 0:
                    continue
                src = (my - d) % N_DEV
                pl.semaphore_signal(
                    credit_sem, inc=1,
                    device_id=(src,), device_id_type=pl.DeviceIdType.MESH,
                )

        for rdma in sends:
            rdma.wait_send()
        pltpu.make_async_copy(acc_ref, out_hbm, out_sems.at[0]).wait()

        pl.semaphore_wait(credit_sem, N_DEV - 1)

    x = pltpu.with_memory_space_constraint(x, pltpu.MemorySpace.HBM)
    w_mat = pltpu.with_memory_space_constraint(w_mat, pltpu.MemorySpace.HBM)
    return pl.pallas_call(
        body,
        out_shape=jax.ShapeDtypeStruct((BLK, n), jnp.float32),
        in_specs=[
            pl.BlockSpec(memory_space=pltpu.MemorySpace.HBM),
            pl.BlockSpec(memory_space=pltpu.MemorySpace.HBM),
        ],
        out_specs=pl.BlockSpec(memory_space=pltpu.MemorySpace.HBM),
        scratch_shapes=[
            pltpu.VMEM((N_DEV * BLK, BLK), jnp.float32),
            pltpu.VMEM((N_DEV * BLK, BLK), jnp.bfloat16),
            pltpu.VMEM((BLK, N_DEV * BLK), jnp.bfloat16),
            pltpu.VMEM((2, 2 * BLK, N_DEV * BLK), jnp.float32),
            pltpu.VMEM((BLK, N_DEV * BLK), jnp.float32),
            pltpu.SemaphoreType.DMA((N_DEV - 1,)),
            pltpu.SemaphoreType.DMA((N_DEV - 1,)),
            pltpu.SemaphoreType.DMA((2, 2)),
            pltpu.SemaphoreType.DMA,
            pltpu.SemaphoreType.DMA((2,)),
            pltpu.SemaphoreType.REGULAR,
        ],
        compiler_params=pltpu.CompilerParams(collective_id=0),
    )(x, w_mat)


# device time: 17276 ns/iter; 1.0583x vs baseline; 1.0396x over previous
import jax
import jax.numpy as jnp
from jax import lax
from jax.experimental import pallas as pl
from jax.experimental.pallas import tpu as pltpu

N_DEV = 8
BLK = 256

_ORDER = [0, 2, 6, 3, 5, 1, 7, 4]
_CHUNKS = [(0, 1), (2, 3), (4, 5), (6, 7)]


def kernel(x, w_mat):
    k, m_per = x.shape
    kw, n = w_mat.shape
    assert m_per == BLK and k == N_DEV * BLK

    def body(x_hbm, w_hbm, out_hbm, xv_ref, xb_ref, gx_ref, wbuf_ref, acc_ref,
             send_sems, recv_sems, w_sems, x_sem, out_sems, credit_sem):
        my = lax.axis_index("i")

        pltpu.make_async_copy(x_hbm, xv_ref, x_sem).start()

        def start_w_chunk(c, slot):
            for h, t in enumerate(_CHUNKS[c]):
                j = (my - _ORDER[t]) % N_DEV
                pltpu.make_async_copy(
                    w_hbm.at[pl.ds(j * BLK, BLK), :],
                    wbuf_ref.at[slot, pl.ds(h * BLK, BLK), :],
                    w_sems.at[slot, h],
                ).start()

        def wait_w_chunk(c, slot):
            for h, t in enumerate(_CHUNKS[c]):
                j = (my - _ORDER[t]) % N_DEV
                pltpu.make_async_copy(
                    w_hbm.at[pl.ds(j * BLK, BLK), :],
                    wbuf_ref.at[slot, pl.ds(h * BLK, BLK), :],
                    w_sems.at[slot, h],
                ).wait()

        start_w_chunk(0, 0)

        barrier = pltpu.get_barrier_semaphore()
        for d in range(1, N_DEV):
            peer = (my + d) % N_DEV
            pl.semaphore_signal(
                barrier, inc=1,
                device_id=(peer,), device_id_type=pl.DeviceIdType.MESH,
            )

        pltpu.make_async_copy(x_hbm, xv_ref, x_sem).wait()
        xb_ref[:, :] = xv_ref[:, :].astype(jnp.bfloat16)

        pl.semaphore_wait(barrier, N_DEV - 1)

        sends = []
        for t in range(1, N_DEV):
            d = _ORDER[t]
            peer = (my + d) % N_DEV
            rdma = pltpu.make_async_remote_copy(
                src_ref=xb_ref.at[pl.ds(peer * BLK, BLK), :],
                dst_ref=gx_ref.at[:, pl.ds(t * BLK, BLK)],
                send_sem=send_sems.at[d - 1],
                recv_sem=recv_sems.at[d - 1],
                device_id=(peer,),
                device_id_type=pl.DeviceIdType.MESH,
            )
            rdma.start()
            sends.append(rdma)

        gx_ref[:, pl.ds(0, BLK)] = xb_ref[pl.ds(my * BLK, BLK), :]

        n_chunks = len(_CHUNKS)
        for c, chunk in enumerate(_CHUNKS):
            slot = c % 2
            if c + 1 < n_chunks:
                start_w_chunk(c + 1, 1 - slot)
            wait_w_chunk(c, slot)
            for t in chunk:
                d = _ORDER[t]
                if d == 0:
                    continue
                recv = pltpu.make_async_remote_copy(
                    src_ref=xb_ref.at[pl.ds(0, BLK), :],
                    dst_ref=gx_ref.at[:, pl.ds(t * BLK, BLK)],
                    send_sem=send_sems.at[d - 1],
                    recv_sem=recv_sems.at[d - 1],
                    device_id=(my,),
                    device_id_type=pl.DeviceIdType.MESH,
                )
                recv.wait_recv()
            kblk = len(chunk) * BLK
            if c + 1 < n_chunks:
                part = jnp.dot(
                    gx_ref[:, pl.ds(chunk[0] * BLK, kblk)],
                    wbuf_ref[slot, pl.ds(0, kblk), :],
                    preferred_element_type=jnp.float32,
                )
                if c == 0:
                    acc_ref[:, :] = part
                else:
                    acc_ref[:, :] = acc_ref[:, :] + part
            else:
                part = jnp.dot(
                    gx_ref[:, pl.ds(chunk[0] * BLK, kblk)],
                    wbuf_ref[slot, pl.ds(0, kblk), :],
                    preferred_element_type=jnp.float32,
                )
                y = acc_ref[:, :] + part
                acc_ref[:, :] = y * jax.nn.sigmoid(y)
                pltpu.make_async_copy(acc_ref, out_hbm, out_sems.at[0]).start()
            for t in chunk:
                d = _ORDER[t]
                if d == 0:
                    continue
                src = (my - d) % N_DEV
                pl.semaphore_signal(
                    credit_sem, inc=1,
                    device_id=(src,), device_id_type=pl.DeviceIdType.MESH,
                )

        for rdma in sends:
            rdma.wait_send()
        pltpu.make_async_copy(acc_ref, out_hbm, out_sems.at[0]).wait()

        pl.semaphore_wait(credit_sem, N_DEV - 1)

    x = pltpu.with_memory_space_constraint(x, pltpu.MemorySpace.HBM)
    w_mat = pltpu.with_memory_space_constraint(w_mat, pltpu.MemorySpace.HBM)
    return pl.pallas_call(
        body,
        out_shape=jax.ShapeDtypeStruct((BLK, n), jnp.float32),
        in_specs=[
            pl.BlockSpec(memory_space=pltpu.MemorySpace.HBM),
            pl.BlockSpec(memory_space=pltpu.MemorySpace.HBM),
        ],
        out_specs=pl.BlockSpec(memory_space=pltpu.MemorySpace.HBM),
        scratch_shapes=[
            pltpu.VMEM((N_DEV * BLK, BLK), jnp.float32),
            pltpu.VMEM((N_DEV * BLK, BLK), jnp.bfloat16),
            pltpu.VMEM((BLK, N_DEV * BLK), jnp.bfloat16),
            pltpu.VMEM((2, 2 * BLK, N_DEV * BLK), jnp.float32),
            pltpu.VMEM((BLK, N_DEV * BLK), jnp.float32),
            pltpu.SemaphoreType.DMA((N_DEV - 1,)),
            pltpu.SemaphoreType.DMA((N_DEV - 1,)),
            pltpu.SemaphoreType.DMA((2, 2)),
            pltpu.SemaphoreType.DMA,
            pltpu.SemaphoreType.DMA((2,)),
            pltpu.SemaphoreType.REGULAR,
        ],
        compiler_params=pltpu.CompilerParams(collective_id=0),
    )(x, w_mat)
